# initial kernel scaffold (unmeasured)
import jax
import jax.numpy as jnp
from jax import lax
from jax.experimental import pallas as pl
from jax.experimental.pallas import tpu as pltpu


def kernel(
    x,
):
    def body(*refs):
        pass

    out_shape = jax.ShapeDtypeStruct(..., jnp.float32)
    return pl.pallas_call(body, out_shape=out_shape)(...)


# baseline (device time: 568525 ns/iter reference)
import jax
import jax.numpy as jnp
from jax import lax
from jax.experimental import pallas as pl
from jax.experimental.pallas import tpu as pltpu


def kernel(x):
    x = x.astype(jnp.bfloat16)
    m, n2 = x.shape
    n = n2 // 2

    def body(x_ref, out_ref, local_sem, send_sem, recv_sem):
        my_x = lax.axis_index("x")
        my_y = lax.axis_index("y")
        my_z = lax.axis_index("z")
        other = 1 - my_x

        barrier = pltpu.get_barrier_semaphore()
        pl.semaphore_signal(
            barrier, inc=1,
            device_id=(other, my_y, my_z),
            device_id_type=pl.DeviceIdType.MESH,
        )
        pl.semaphore_wait(barrier, 1)

        rdma = pltpu.make_async_remote_copy(
            src_ref=x_ref.at[:, pl.ds(other * n, n)],
            dst_ref=out_ref.at[pl.ds(my_x * m, m), :],
            send_sem=send_sem,
            recv_sem=recv_sem,
            device_id=(other, my_y, my_z),
            device_id_type=pl.DeviceIdType.MESH,
        )
        rdma.start()

        copy = pltpu.make_async_copy(
            x_ref.at[:, pl.ds(my_x * n, n)],
            out_ref.at[pl.ds(my_x * m, m), :],
            local_sem,
        )
        copy.start()
        copy.wait()

        rdma.wait()

    return pl.pallas_call(
        body,
        out_shape=jax.ShapeDtypeStruct((2 * m, n), jnp.bfloat16),
        in_specs=[pl.BlockSpec(memory_space=pltpu.MemorySpace.HBM)],
        out_specs=pl.BlockSpec(memory_space=pltpu.MemorySpace.HBM),
        scratch_shapes=[
            pltpu.SemaphoreType.DMA,
            pltpu.SemaphoreType.DMA,
            pltpu.SemaphoreType.DMA,
        ],
        compiler_params=pltpu.CompilerParams(collective_id=0),
    )(x)


# device time: 265428 ns/iter; 2.1419x vs baseline; 2.1419x over previous
import jax
import jax.numpy as jnp
from jax import lax
from jax.experimental import pallas as pl
from jax.experimental.pallas import tpu as pltpu


def kernel(x):
    m, n2 = x.shape
    n = n2 // 2
    my_x = lax.axis_index("x")
    other = 1 - my_x

    xb = x.astype(jnp.bfloat16)
    x_keep = lax.dynamic_slice(xb, (0, my_x * n), (m, n))
    x_send = lax.dynamic_slice(xb, (0, other * n), (m, n))

    def body(keep_ref, send_ref, out_ref, local_sem, send_sem, recv_sem):
        my_x = lax.axis_index("x")
        my_y = lax.axis_index("y")
        my_z = lax.axis_index("z")
        other = 1 - my_x

        barrier = pltpu.get_barrier_semaphore()
        pl.semaphore_signal(
            barrier, inc=1,
            device_id=(other, my_y, my_z),
            device_id_type=pl.DeviceIdType.MESH,
        )
        pl.semaphore_wait(barrier, 1)

        rdma = pltpu.make_async_remote_copy(
            src_ref=send_ref,
            dst_ref=out_ref.at[pl.ds(my_x * m, m), :],
            send_sem=send_sem,
            recv_sem=recv_sem,
            device_id=(other, my_y, my_z),
            device_id_type=pl.DeviceIdType.MESH,
        )
        rdma.start()

        copy = pltpu.make_async_copy(
            keep_ref,
            out_ref.at[pl.ds(my_x * m, m), :],
            local_sem,
        )
        copy.start()
        copy.wait()

        rdma.wait()

    return pl.pallas_call(
        body,
        out_shape=jax.ShapeDtypeStruct((2 * m, n), jnp.bfloat16),
        in_specs=[
            pl.BlockSpec(memory_space=pltpu.MemorySpace.HBM),
            pl.BlockSpec(memory_space=pltpu.MemorySpace.HBM),
        ],
        out_specs=pl.BlockSpec(memory_space=pltpu.MemorySpace.HBM),
        scratch_shapes=[
            pltpu.SemaphoreType.DMA,
            pltpu.SemaphoreType.DMA,
            pltpu.SemaphoreType.DMA,
        ],
        compiler_params=pltpu.CompilerParams(collective_id=0),
    )(x_keep, x_send)


# device time: 207998 ns/iter; 2.7333x vs baseline; 1.2761x over previous
import jax
import jax.numpy as jnp
from jax import lax
from jax.experimental import pallas as pl
from jax.experimental.pallas import tpu as pltpu

R = 512
S = 4


def kernel(x):
    m, n2 = x.shape
    n = n2 // 2
    K = m // R

    def body(x_ref, out_ref, load_buf, send_buf, keep_buf,
             load_sems, send_sems, recv_sems, keep_sems):
        my_x = lax.axis_index("x")
        my_y = lax.axis_index("y")
        my_z = lax.axis_index("z")
        other = 1 - my_x

        def load(k):
            c = pltpu.make_async_copy(
                x_ref.at[pl.ds(k * R, R), :], load_buf.at[k % 2],
                load_sems.at[k % 2])
            c.start()
            return c

        def rdma(k):
            return pltpu.make_async_remote_copy(
                src_ref=send_buf.at[k % S],
                dst_ref=out_ref.at[pl.ds(my_x * m + k * R, R), :],
                send_sem=send_sems.at[k],
                recv_sem=recv_sems.at[k],
                device_id=(other, my_y, my_z),
                device_id_type=pl.DeviceIdType.MESH,
            )

        def keep_dma(k):
            return pltpu.make_async_copy(
                keep_buf.at[k % S],
                out_ref.at[pl.ds(my_x * m + k * R, R), :],
                keep_sems.at[k])

        loads = {0: load(0), 1: load(1)}
        rdmas = {}
        keeps = {}

        barrier = pltpu.get_barrier_semaphore()
        pl.semaphore_signal(
            barrier, inc=1, device_id=(other, my_y, my_z),
            device_id_type=pl.DeviceIdType.MESH)
        pl.semaphore_wait(barrier, 1)

        for k in range(K):
            loads[k].wait()
            if k >= S:
                rdmas[k - S].wait_send()
                keeps[k - S].wait()
            @pl.when(my_x == 0)
            def _():
                keep_buf[k % S] = load_buf[k % 2, :, 0:n].astype(jnp.bfloat16)
                send_buf[k % S] = load_buf[k % 2, :, n:n2].astype(jnp.bfloat16)

            @pl.when(my_x == 1)
            def _():
                keep_buf[k % S] = load_buf[k % 2, :, n:n2].astype(jnp.bfloat16)
                send_buf[k % S] = load_buf[k % 2, :, 0:n].astype(jnp.bfloat16)

            rdmas[k] = rdma(k)
            rdmas[k].start()
            keeps[k] = keep_dma(k)
            keeps[k].start()
            if k + 2 < K:
                loads[k + 2] = load(k + 2)

        for k in range(max(0, K - S), K):
            rdmas[k].wait_send()
            keeps[k].wait()
        for k in range(K):
            rdmas[k].wait_recv()

    return pl.pallas_call(
        body,
        out_shape=jax.ShapeDtypeStruct((2 * m, n), jnp.bfloat16),
        in_specs=[pl.BlockSpec(memory_space=pltpu.MemorySpace.HBM)],
        out_specs=pl.BlockSpec(memory_space=pltpu.MemorySpace.HBM),
        scratch_shapes=[
            pltpu.VMEM((2, R, n2), jnp.float32),
            pltpu.VMEM((S, R, n), jnp.bfloat16),
            pltpu.VMEM((S, R, n), jnp.bfloat16),
            pltpu.SemaphoreType.DMA((2,)),
            pltpu.SemaphoreType.DMA((K,)),
            pltpu.SemaphoreType.DMA((K,)),
            pltpu.SemaphoreType.DMA((K,)),
        ],
        compiler_params=pltpu.CompilerParams(collective_id=0),
    )(x)
